# baseline (speedup 1.0000x reference)
import functools

import jax
import jax.numpy as jnp
from jax import lax
from jax.experimental import pallas as pl
from jax.experimental.pallas import tpu as pltpu
from jax.experimental.pallas import tpu_sc as plsc

B = 4096
S = 200
E = 128
V = 100000
H = 512
O = 2

NC, NS = 2, 16
NW = NC * NS
BPW = B // NW
CHUNK = 128
NPOS = BPW * S
NCHUNK = NPOS // CHUNK
NBUF = 4
LANES = 16


def _sc_mean(idx, table):
    mesh = plsc.VectorSubcoreMesh(core_axis_name="c", subcore_axis_name="s")

    @functools.partial(
        pl.kernel,
        mesh=mesh,
        out_type=jax.ShapeDtypeStruct((B, E), jnp.float32),
        scratch_types=[
            pltpu.VMEM((NPOS,), jnp.int32),
            pltpu.VMEM((NBUF * CHUNK,), jnp.int32),
            pltpu.VMEM((CHUNK, E), jnp.float32),
            pltpu.VMEM((CHUNK, E), jnp.float32),
            pltpu.VMEM((CHUNK, E), jnp.float32),
            pltpu.VMEM((CHUNK, E), jnp.float32),
            pltpu.VMEM_SHARED((NS * BPW, E), jnp.float32),
            pltpu.SemaphoreType.DMA,
            pltpu.SemaphoreType.DMA,
            pltpu.SemaphoreType.DMA,
            pltpu.SemaphoreType.DMA,
            pltpu.SemaphoreType.DMA,
            pltpu.SemaphoreType.DMA,
            pltpu.SemaphoreType.DMA,
            pltpu.SemaphoreType.DMA,
        ],
    )
    def k(table_hbm, idx_hbm, out_hbm, idx_v, imap, b0, b1, b2, b3, acc,
          sg0, sg1, sg2, sg3, ss0, ss1, ss2, ss3):
        sid = lax.axis_index("s")
        cid = lax.axis_index("c")
        wid = sid * NC + cid
        base = wid * BPW
        bufs = (b0, b1, b2, b3)
        gsems = (sg0, sg1, sg2, sg3)
        ssems = (ss0, ss1, ss2, ss3)

        pltpu.sync_copy(idx_hbm.at[pl.ds(base * S, NPOS)], idx_v)

        zero = jnp.zeros((LANES,), jnp.float32)

        @pl.loop(0, BPW)
        def _(r):
            for c in range(E // LANES):
                b0[r, pl.ds(c * LANES, LANES)] = zero

        pltpu.sync_copy(b0, acc.at[pl.ds(sid * BPW, BPW)])

        slot_base = sid * BPW
        pos_iota = lax.iota(jnp.int32, LANES)

        def build_imap(kk, j):
            for j8 in range(CHUNK // LANES):
                p = kk * CHUNK + j8 * LANES + pos_iota
                imap[pl.ds(j * CHUNK + j8 * LANES, LANES)] = (
                    slot_base + lax.div(p, S))

        def gather(kk, buf, sem):
            off = pl.multiple_of(kk * CHUNK, 8)
            pltpu.async_copy(table_hbm.at[idx_v.at[pl.ds(off, CHUNK)]], buf,
                             sem)

        def wait_g(buf, sem):
            pltpu.make_async_copy(table_hbm.at[pl.ds(0, CHUNK)], buf,
                                  sem).wait()

        def scat(j, buf, sem):
            pltpu.async_copy(buf, acc.at[imap.at[pl.ds(j * CHUNK, CHUNK)]],
                             sem, add=True)

        def wait_s(buf, sem):
            pltpu.make_async_copy(buf, acc.at[pl.ds(0, CHUNK)], sem).wait()

        for j in range(NBUF):
            gather(j, bufs[j], gsems[j])

        @pl.loop(0, NCHUNK, step=NBUF)
        def _(kk):
            for j in range(NBUF):
                build_imap(kk + j, j)
                wait_g(bufs[j], gsems[j])
                scat(j, bufs[j], ssems[j])
            for j in range(NBUF):
                wait_s(bufs[j], ssems[j])
                nxt = jnp.where(kk + NBUF + j >= NCHUNK, j, kk + NBUF + j)
                gather(nxt, bufs[j], gsems[j])

        for j in range(NBUF):
            wait_g(bufs[j], gsems[j])

        pltpu.sync_copy(acc.at[pl.ds(sid * BPW, BPW)], b0)

        @pl.loop(0, BPW)
        def _(r):
            for c in range(E // LANES):
                b0[r, pl.ds(c * LANES, LANES)] = (
                    b0[r, pl.ds(c * LANES, LANES)] * (1.0 / S))

        pltpu.sync_copy(b0, out_hbm.at[pl.ds(base, BPW)])

    return k(table, idx)


def _tc_mlp(avg, W1, b1, W2, b2):
    BB = 512

    def body(x_ref, w1_ref, b1_ref, w2_ref, b2_ref, o_ref):
        x = x_ref[...]
        h = jnp.dot(x, w1_ref[...], preferred_element_type=jnp.float32)
        h = jnp.maximum(h + b1_ref[...], 0.0)
        logits = jnp.dot(h, w2_ref[...], preferred_element_type=jnp.float32)
        logits = logits + b2_ref[...]
        m = jnp.max(logits, axis=-1, keepdims=True)
        e = jnp.exp(logits - m)
        lse = m + jnp.log(jnp.sum(e, axis=-1, keepdims=True))
        o_ref[...] = logits - lse

    return pl.pallas_call(
        body,
        grid=(B // BB,),
        in_specs=[
            pl.BlockSpec((BB, E), lambda i: (i, 0)),
            pl.BlockSpec((E, H), lambda i: (0, 0)),
            pl.BlockSpec((1, H), lambda i: (0, 0)),
            pl.BlockSpec((H, O), lambda i: (0, 0)),
            pl.BlockSpec((1, O), lambda i: (0, 0)),
        ],
        out_specs=pl.BlockSpec((BB, O), lambda i: (i, 0)),
        out_shape=jax.ShapeDtypeStruct((B, O), jnp.float32),
    )(avg, W1, b1.reshape(1, H), W2, b2.reshape(1, O))


def kernel(word_indices, table, W1, b1, W2, b2):
    idx = word_indices.astype(jnp.int32).reshape(B * S)
    avg = _sc_mean(idx, table)
    return _tc_mlp(avg, W1, b1, W2, b2)


# speedup vs baseline: 1.7179x; 1.7179x over previous
import functools

import jax
import jax.numpy as jnp
from jax import lax
from jax.experimental import pallas as pl
from jax.experimental.pallas import tpu as pltpu
from jax.experimental.pallas import tpu_sc as plsc

B = 4096
S = 200
E = 128
V = 100000
H = 512
O = 2

NC, NS = 2, 16
NW = NC * NS
BPW = B // NW
CH0 = 104
CH1 = S - CH0
NPAIR = 3
LOOPR = BPW - BPW % NPAIR
LANES = 16


def _sc_mean(idx, table):
    mesh = plsc.VectorSubcoreMesh(core_axis_name="c", subcore_axis_name="s")

    @functools.partial(
        pl.kernel,
        mesh=mesh,
        out_type=jax.ShapeDtypeStruct((B, E), jnp.float32),
        scratch_types=[
            pltpu.VMEM((BPW * S,), jnp.int32),
            pltpu.VMEM((CH0, E), jnp.float32),
            pltpu.VMEM((CH1, E), jnp.float32),
            pltpu.VMEM((CH0, E), jnp.float32),
            pltpu.VMEM((CH1, E), jnp.float32),
            pltpu.VMEM((CH0, E), jnp.float32),
            pltpu.VMEM((CH1, E), jnp.float32),
            pltpu.VMEM((BPW, E), jnp.float32),
            pltpu.SemaphoreType.DMA,
            pltpu.SemaphoreType.DMA,
            pltpu.SemaphoreType.DMA,
        ],
    )
    def k(table_hbm, idx_hbm, out_hbm, idx_v, ra0, ra1, rb0, rb1, rc0, rc1,
          out_v, sem_a, sem_b, sem_c):
        wid = lax.axis_index("s") * NC + lax.axis_index("c")
        base = wid * BPW
        pairs = ((ra0, ra1, sem_a), (rb0, rb1, sem_b), (rc0, rc1, sem_c))

        pltpu.sync_copy(idx_hbm.at[pl.ds(base * S, BPW * S)], idx_v)

        def issue(b, pair):
            r0, r1, sem = pair
            off = pl.multiple_of(b * S, 8)
            pltpu.async_copy(table_hbm.at[idx_v.at[pl.ds(off, CH0)]], r0, sem)
            pltpu.async_copy(
                table_hbm.at[idx_v.at[pl.ds(off + CH0, CH1)]], r1, sem)

        def wait(pair):
            r0, r1, sem = pair
            pltpu.make_async_copy(table_hbm.at[pl.ds(0, CH0)], r0, sem).wait()
            pltpu.make_async_copy(table_hbm.at[pl.ds(0, CH1)], r1, sem).wait()

        def reduce_into(b, pair):
            r0, r1, _ = pair

            def body0(r, accs):
                return tuple(
                    accs[c] + r0[r, pl.ds(c * LANES, LANES)]
                    for c in range(E // LANES)
                )

            def body1(r, accs):
                return tuple(
                    accs[c] + r1[r, pl.ds(c * LANES, LANES)]
                    for c in range(E // LANES)
                )

            accs = tuple(
                jnp.zeros((LANES,), jnp.float32) for _ in range(E // LANES))
            accs = lax.fori_loop(0, CH0, body0, accs)
            accs = lax.fori_loop(0, CH1, body1, accs)
            for c in range(E // LANES):
                out_v[b, pl.ds(c * LANES, LANES)] = accs[c] * (1.0 / S)

        for j in range(NPAIR):
            issue(j, pairs[j])

        @pl.loop(0, LOOPR, step=NPAIR)
        def _(b):
            for j in range(NPAIR):
                wait(pairs[j])
                reduce_into(b + j, pairs[j])
                raw = b + NPAIR + j
                nxt = jnp.where(raw >= BPW, j, raw)
                issue(nxt, pairs[j])

        for j in range(NPAIR):
            wait(pairs[j])
            if j < BPW % NPAIR:
                reduce_into(LOOPR + j, pairs[j])

        pltpu.sync_copy(out_v, out_hbm.at[pl.ds(base, BPW)])

    return k(table, idx)


def _tc_mlp(avg, W1, b1, W2, b2):
    BB = 512

    def body(x_ref, w1_ref, b1_ref, w2_ref, b2_ref, o_ref):
        x = x_ref[...]
        h = jnp.dot(x, w1_ref[...], preferred_element_type=jnp.float32)
        h = jnp.maximum(h + b1_ref[...], 0.0)
        logits = jnp.dot(h, w2_ref[...], preferred_element_type=jnp.float32)
        logits = logits + b2_ref[...]
        m = jnp.max(logits, axis=-1, keepdims=True)
        e = jnp.exp(logits - m)
        lse = m + jnp.log(jnp.sum(e, axis=-1, keepdims=True))
        o_ref[...] = logits - lse

    return pl.pallas_call(
        body,
        grid=(B // BB,),
        in_specs=[
            pl.BlockSpec((BB, E), lambda i: (i, 0)),
            pl.BlockSpec((E, H), lambda i: (0, 0)),
            pl.BlockSpec((1, H), lambda i: (0, 0)),
            pl.BlockSpec((H, O), lambda i: (0, 0)),
            pl.BlockSpec((1, O), lambda i: (0, 0)),
        ],
        out_specs=pl.BlockSpec((BB, O), lambda i: (i, 0)),
        out_shape=jax.ShapeDtypeStruct((B, O), jnp.float32),
    )(avg, W1, b1.reshape(1, H), W2, b2.reshape(1, O))


def kernel(word_indices, table, W1, b1, W2, b2):
    idx = word_indices.astype(jnp.int32).reshape(B * S)
    avg = _sc_mean(idx, table)
    return _tc_mlp(avg, W1, b1, W2, b2)


# speedup vs baseline: 1.7228x; 1.0028x over previous
import functools

import jax
import jax.numpy as jnp
from jax import lax
from jax.experimental import pallas as pl
from jax.experimental.pallas import tpu as pltpu
from jax.experimental.pallas import tpu_sc as plsc

B = 4096
S = 200
E = 128
V = 100000
H = 512
O = 2

NC, NS = 2, 16
NW = NC * NS
BPW = B // NW
CHS = (72, 64, 64)
OFFS = (0, 72, 136)
NPAIR = 3
LOOPR = BPW - BPW % NPAIR
LANES = 16


def _sc_mean(idx, table):
    mesh = plsc.VectorSubcoreMesh(core_axis_name="c", subcore_axis_name="s")

    @functools.partial(
        pl.kernel,
        mesh=mesh,
        out_type=jax.ShapeDtypeStruct((B, E), jnp.float32),
        scratch_types=[
            pltpu.VMEM((BPW * S,), jnp.int32),
            pltpu.VMEM((CHS[0], E), jnp.float32),
            pltpu.VMEM((CHS[1], E), jnp.float32),
            pltpu.VMEM((CHS[2], E), jnp.float32),
            pltpu.VMEM((CHS[0], E), jnp.float32),
            pltpu.VMEM((CHS[1], E), jnp.float32),
            pltpu.VMEM((CHS[2], E), jnp.float32),
            pltpu.VMEM((CHS[0], E), jnp.float32),
            pltpu.VMEM((CHS[1], E), jnp.float32),
            pltpu.VMEM((CHS[2], E), jnp.float32),
            pltpu.VMEM((BPW, E), jnp.float32),
            pltpu.SemaphoreType.DMA,
            pltpu.SemaphoreType.DMA,
            pltpu.SemaphoreType.DMA,
        ],
    )
    def k(table_hbm, idx_hbm, out_hbm, idx_v, ra0, ra1, ra2, rb0, rb1, rb2,
          rc0, rc1, rc2, out_v, sem_a, sem_b, sem_c):
        wid = lax.axis_index("s") * NC + lax.axis_index("c")
        base = wid * BPW
        pairs = ((ra0, ra1, ra2, sem_a), (rb0, rb1, rb2, sem_b),
                 (rc0, rc1, rc2, sem_c))

        pltpu.sync_copy(idx_hbm.at[pl.ds(base * S, BPW * S)], idx_v)

        def issue(b, pair):
            sem = pair[3]
            off = pl.multiple_of(b * S, 8)
            for t in range(3):
                pltpu.async_copy(
                    table_hbm.at[idx_v.at[pl.ds(off + OFFS[t], CHS[t])]],
                    pair[t], sem)

        def wait(pair):
            sem = pair[3]
            for t in range(3):
                pltpu.make_async_copy(table_hbm.at[pl.ds(0, CHS[t])],
                                      pair[t], sem).wait()

        def reduce_into(b, pair):
            accs = tuple(
                jnp.zeros((LANES,), jnp.float32) for _ in range(E // LANES))
            for t in range(3):
                rt = pair[t]

                def body(r, accs, rt=rt):
                    return tuple(
                        accs[c] + rt[r, pl.ds(c * LANES, LANES)]
                        for c in range(E // LANES)
                    )

                accs = lax.fori_loop(0, CHS[t], body, accs)
            for c in range(E // LANES):
                out_v[b, pl.ds(c * LANES, LANES)] = accs[c] * (1.0 / S)

        for j in range(NPAIR):
            issue(j, pairs[j])

        @pl.loop(0, LOOPR, step=NPAIR)
        def _(b):
            for j in range(NPAIR):
                wait(pairs[j])
                reduce_into(b + j, pairs[j])
                raw = b + NPAIR + j
                nxt = jnp.where(raw >= BPW, j, raw)
                issue(nxt, pairs[j])

        for j in range(NPAIR):
            wait(pairs[j])
            if j < BPW % NPAIR:
                reduce_into(LOOPR + j, pairs[j])

        pltpu.sync_copy(out_v, out_hbm.at[pl.ds(base, BPW)])

    return k(table, idx)


def _tc_mlp(avg, W1, b1, W2, b2):
    BB = 512

    def body(x_ref, w1_ref, b1_ref, w2_ref, b2_ref, o_ref):
        x = x_ref[...]
        h = jnp.dot(x, w1_ref[...], preferred_element_type=jnp.float32)
        h = jnp.maximum(h + b1_ref[...], 0.0)
        logits = jnp.dot(h, w2_ref[...], preferred_element_type=jnp.float32)
        logits = logits + b2_ref[...]
        m = jnp.max(logits, axis=-1, keepdims=True)
        e = jnp.exp(logits - m)
        lse = m + jnp.log(jnp.sum(e, axis=-1, keepdims=True))
        o_ref[...] = logits - lse

    return pl.pallas_call(
        body,
        grid=(B // BB,),
        in_specs=[
            pl.BlockSpec((BB, E), lambda i: (i, 0)),
            pl.BlockSpec((E, H), lambda i: (0, 0)),
            pl.BlockSpec((1, H), lambda i: (0, 0)),
            pl.BlockSpec((H, O), lambda i: (0, 0)),
            pl.BlockSpec((1, O), lambda i: (0, 0)),
        ],
        out_specs=pl.BlockSpec((BB, O), lambda i: (i, 0)),
        out_shape=jax.ShapeDtypeStruct((B, O), jnp.float32),
    )(avg, W1, b1.reshape(1, H), W2, b2.reshape(1, O))


def kernel(word_indices, table, W1, b1, W2, b2):
    idx = word_indices.astype(jnp.int32).reshape(B * S)
    avg = _sc_mean(idx, table)
    return _tc_mlp(avg, W1, b1, W2, b2)
